# Initial kernel scaffold; baseline (speedup 1.0000x reference)
import jax
import jax.numpy as jnp
from jax.experimental import pallas as pl


def kernel(x):
    raise NotImplementedError("write your pallas kernel here")


# speedup vs baseline: 5.0997x; 5.0997x over previous
import functools

import jax
import jax.numpy as jnp
from jax import lax
from jax.experimental import pallas as pl

OP_SHL = 0
OP_SHR = 1
MARK_AX = 2
ALU_LO = 16
ALU_HI = 32
AX_CARRY_LO = 48
OUTPUT_LO = 64
OUTPUT_HI = 80

MAGIC32 = 1.5 * float(2 ** 23)


def _magic_floor(x):
    return (x - 0.5 + 0.001) + MAGIC32 - MAGIC32


def _tc_body(x_ref, o_ref):
    xb = x_ref[...]
    br = xb.shape[0]
    li = lax.broadcasted_iota(jnp.int32, (br, 512), 1)

    def argmax16(start):
        mask = (li >= start) & (li < start + 16)
        vals = jnp.where(mask, xb, -1.0)
        m = jnp.max(vals, axis=1, keepdims=True)
        cand = jnp.where(mask & (xb == m), li, 512)
        return jnp.min(cand, axis=1, keepdims=True) - start

    a_lo = argmax16(ALU_LO)
    a_hi = argmax16(ALU_HI)
    sh = argmax16(AX_CARRY_LO)

    op_shl = xb[:, OP_SHL:OP_SHL + 1]
    op_shr = xb[:, OP_SHR:OP_SHR + 1]
    mark_ax = xb[:, MARK_AX:MARK_AX + 1]
    active_shl = (op_shl > 0.5) & (mark_ax > 0.5)
    active_shr = (op_shr > 0.5) & (mark_ax > 0.5)

    a = a_lo + 16 * a_hi
    shv = jnp.minimum(sh, 7)
    hi_shr = lax.shift_right_logical(a, shv + 4)

    idx_lo = jnp.full_like(a, OUTPUT_LO)
    idx_hi = OUTPUT_HI + jnp.where(active_shl, 0, hi_shr)
    active_f = (active_shl | active_shr).astype(jnp.float32)

    delta = jnp.where(li == idx_lo, active_f, 0.0) + jnp.where(
        li == idx_hi, active_f, 0.0)
    o_ref[...] = xb + delta


@functools.partial(jax.jit, static_argnames=("interpret",))
def kernel(x, interpret=False):
    B, D = x.shape
    BR = 1024
    return pl.pallas_call(
        _tc_body,
        grid=(B // BR,),
        in_specs=[pl.BlockSpec((BR, D), lambda i: (i, 0))],
        out_specs=pl.BlockSpec((BR, D), lambda i: (i, 0)),
        out_shape=jax.ShapeDtypeStruct((B, D), x.dtype),
        interpret=interpret,
    )(x)
